# baseline (speedup 1.0000x reference)
import jax
import jax.numpy as jnp
from jax import lax
from jax.experimental import pallas as pl
from jax.experimental.pallas import tpu as pltpu
from jax.experimental.pallas import tpu_sc as plsc

GAMMA = 0.1
EPS = 1e-8
N = 100000
D = 256
TOPK = 5

BLOCK = 16384
NBLK = 7
NPAD = NBLK * BLOCK
ROWS = NPAD // 128

NW = 32
BPW = NPAD // NW
LANES = 16
CHUNKS = BPW // LANES

TROWS = 8
TCOLS = N // TROWS


def _sim_body_floor(nf_ref, inj_ref, out_ref):
    out_ref[...] = jnp.zeros((1, 1, BLOCK), jnp.float32) + nf_ref[0, 0]


def _sim_body(nf_ref, inj_ref, out_ref):
    x = nf_ref[...]
    inj = inj_ref[...]
    dims = (((1,), (1,)), ((), ()))
    num = lax.dot_general(inj, x, dims,
                          preferred_element_type=jnp.float32)
    sq = lax.dot_general(jnp.ones((1, D), jnp.float32), x * x, dims,
                         preferred_element_type=jnp.float32)
    inj_norm = jnp.maximum(jnp.sqrt(jnp.sum(inj * inj)), EPS)
    sim = num / (jnp.maximum(jnp.sqrt(sq), EPS) * inj_norm)
    out_ref[...] = sim.reshape(1, 1, BLOCK)


def _node_sim(node_features, inj2d, interpret=False):
    return pl.pallas_call(
        _sim_body,
        grid=(NBLK,),
        in_specs=[
            pl.BlockSpec((BLOCK, D), lambda i: (i, 0)),
            pl.BlockSpec((1, D), lambda i: (0, 0)),
        ],
        out_specs=pl.BlockSpec((1, 1, BLOCK), lambda i: (i, 0, 0)),
        out_shape=jax.ShapeDtypeStruct((NBLK, 1, BLOCK), jnp.float32),
        interpret=interpret,
    )(node_features, inj2d)


def _edge_body(sim_hbm, idx_hbm, infl_hbm, out_hbm,
               idx_v, val_v, infl_v, out_v, sem):
    wid = lax.axis_index("s") * 2 + lax.axis_index("c")
    base = pl.multiple_of(jnp.minimum(wid * BPW, N - BPW), 8)
    pltpu.sync_copy(idx_hbm.at[pl.ds(base, BPW)], idx_v)
    gather = pltpu.async_copy(sim_hbm.at[idx_v], val_v, sem)
    pltpu.sync_copy(infl_hbm.at[pl.ds(base, BPW)], infl_v)
    gather.wait()

    def body(i, c):
        s = pl.ds(i * LANES, LANES)
        out_v[s] = infl_v[s] - GAMMA * val_v[s]
        return c

    lax.fori_loop(0, CHUNKS, body, 0, unroll=4)
    pltpu.sync_copy(out_v, out_hbm.at[pl.ds(base, BPW)])


def _edge_scores(sim_flat, idx, infl):
    mesh = plsc.VectorSubcoreMesh(core_axis_name="c", subcore_axis_name="s")
    k = pl.kernel(
        _edge_body,
        out_type=jax.ShapeDtypeStruct((N,), jnp.float32),
        mesh=mesh,
        compiler_params=pltpu.CompilerParams(needs_layout_passes=False),
        scratch_types=[
            pltpu.VMEM((BPW,), jnp.int32),
            pltpu.VMEM((BPW,), jnp.float32),
            pltpu.VMEM((BPW,), jnp.float32),
            pltpu.VMEM((BPW,), jnp.float32),
            pltpu.SemaphoreType.DMA,
        ],
    )
    return k(sim_flat, idx, infl)


def _topk_body(score_ref, cand_ref, out_ref):
    s = score_ref[...]
    cand = cand_ref[...]
    r = lax.broadcasted_iota(jnp.int32, (TROWS, TCOLS), 0)
    c = lax.broadcasted_iota(jnp.int32, (TROWS, TCOLS), 1)
    flat = r * TCOLS + c
    big = jnp.int32(2**31 - 1)
    neg = jnp.float32(-jnp.inf)
    for i in range(TOPK):
        m = jnp.max(s)
        fi = jnp.min(jnp.where(s == m, flat, big))
        hit = flat == fi
        out_ref[i] = jnp.max(jnp.where(hit, cand, -1))
        s = jnp.where(hit, neg, s)


def _topk(scores2d, cand2d, interpret=False):
    return pl.pallas_call(
        _topk_body,
        in_specs=[
            pl.BlockSpec(memory_space=pltpu.VMEM),
            pl.BlockSpec(memory_space=pltpu.VMEM),
        ],
        out_specs=pl.BlockSpec(memory_space=pltpu.SMEM),
        out_shape=jax.ShapeDtypeStruct((TOPK,), jnp.int32),
        interpret=interpret,
    )(scores2d, cand2d)


def kernel(influence_scores, node_features, candidate_indices, injected_feat):
    idx = candidate_indices.astype(jnp.int32)
    sim = _node_sim(node_features, injected_feat.reshape(1, D)).reshape(NPAD)
    edge = _edge_scores(sim, idx, influence_scores)
    sel = _topk(edge.reshape(TROWS, TCOLS), idx.reshape(TROWS, TCOLS))
    return sel, edge


# speedup vs baseline: 1.0458x; 1.0458x over previous
import jax
import jax.numpy as jnp
from jax import lax
from jax.experimental import pallas as pl
from jax.experimental.pallas import tpu as pltpu
from jax.experimental.pallas import tpu_sc as plsc

GAMMA = 0.1
EPS = 1e-8
N = 100000
D = 256
TOPK = 5

BLOCK = 10240
NBLK = 10
NPAD = NBLK * BLOCK
ROWS = NPAD // 128

NW = 32
BPW = NPAD // NW
LANES = 16
CHUNKS = BPW // LANES

TROWS = 8
TCOLS = N // TROWS


def _sim_body_floor(nf_ref, inj_ref, out_ref):
    out_ref[...] = jnp.zeros((1, 1, BLOCK), jnp.float32) + nf_ref[0, 0]


def _sim_body(nf_ref, inj_ref, out_ref):
    x = nf_ref[...]
    inj = inj_ref[...]
    dims = (((1,), (1,)), ((), ()))
    num = lax.dot_general(inj, x, dims,
                          preferred_element_type=jnp.float32)
    sq = lax.dot_general(jnp.ones((1, D), jnp.float32), x * x, dims,
                         preferred_element_type=jnp.float32)
    inj_norm = jnp.maximum(jnp.sqrt(jnp.sum(inj * inj)), EPS)
    sim = num / (jnp.maximum(jnp.sqrt(sq), EPS) * inj_norm)
    out_ref[...] = sim.reshape(1, 1, BLOCK)


def _node_sim(node_features, inj2d, interpret=False):
    return pl.pallas_call(
        _sim_body,
        grid=(NBLK,),
        in_specs=[
            pl.BlockSpec((BLOCK, D), lambda i: (i, 0)),
            pl.BlockSpec((1, D), lambda i: (0, 0)),
        ],
        out_specs=pl.BlockSpec((1, 1, BLOCK), lambda i: (i, 0, 0)),
        out_shape=jax.ShapeDtypeStruct((NBLK, 1, BLOCK), jnp.float32),
        interpret=interpret,
    )(node_features, inj2d)


def _edge_body(sim_hbm, idx_hbm, infl_hbm, out_hbm,
               idx_v, val_v, infl_v, out_v, sem):
    wid = lax.axis_index("s") * 2 + lax.axis_index("c")
    base = pl.multiple_of(jnp.minimum(wid * BPW, N - BPW), 8)
    pltpu.sync_copy(idx_hbm.at[pl.ds(base, BPW)], idx_v)
    gather = pltpu.async_copy(sim_hbm.at[idx_v], val_v, sem)
    pltpu.sync_copy(infl_hbm.at[pl.ds(base, BPW)], infl_v)
    gather.wait()

    def body(i, c):
        s = pl.ds(i * LANES, LANES)
        out_v[s] = infl_v[s] - GAMMA * val_v[s]
        return c

    lax.fori_loop(0, CHUNKS, body, 0, unroll=4)
    pltpu.sync_copy(out_v, out_hbm.at[pl.ds(base, BPW)])


def _edge_scores(sim_flat, idx, infl):
    mesh = plsc.VectorSubcoreMesh(core_axis_name="c", subcore_axis_name="s")
    k = pl.kernel(
        _edge_body,
        out_type=jax.ShapeDtypeStruct((N,), jnp.float32),
        mesh=mesh,
        compiler_params=pltpu.CompilerParams(needs_layout_passes=False),
        scratch_types=[
            pltpu.VMEM((BPW,), jnp.int32),
            pltpu.VMEM((BPW,), jnp.float32),
            pltpu.VMEM((BPW,), jnp.float32),
            pltpu.VMEM((BPW,), jnp.float32),
            pltpu.SemaphoreType.DMA,
        ],
    )
    return k(sim_flat, idx, infl)


def _topk_body(score_ref, cand_ref, out_ref):
    s = score_ref[...]
    cand = cand_ref[...]
    r = lax.broadcasted_iota(jnp.int32, (TROWS, TCOLS), 0)
    c = lax.broadcasted_iota(jnp.int32, (TROWS, TCOLS), 1)
    flat = r * TCOLS + c
    big = jnp.int32(2**31 - 1)
    neg = jnp.float32(-jnp.inf)
    for i in range(TOPK):
        m = jnp.max(s)
        fi = jnp.min(jnp.where(s == m, flat, big))
        hit = flat == fi
        out_ref[i] = jnp.max(jnp.where(hit, cand, -1))
        s = jnp.where(hit, neg, s)


def _topk(scores2d, cand2d, interpret=False):
    return pl.pallas_call(
        _topk_body,
        in_specs=[
            pl.BlockSpec(memory_space=pltpu.VMEM),
            pl.BlockSpec(memory_space=pltpu.VMEM),
        ],
        out_specs=pl.BlockSpec(memory_space=pltpu.SMEM),
        out_shape=jax.ShapeDtypeStruct((TOPK,), jnp.int32),
        interpret=interpret,
    )(scores2d, cand2d)


def kernel(influence_scores, node_features, candidate_indices, injected_feat):
    idx = candidate_indices.astype(jnp.int32)
    sim = _node_sim(node_features, injected_feat.reshape(1, D)).reshape(NPAD)
    edge = _edge_scores(sim, idx, influence_scores)
    sel = _topk(edge.reshape(TROWS, TCOLS), idx.reshape(TROWS, TCOLS))
    return sel, edge


# speedup vs baseline: 1.0502x; 1.0042x over previous
import jax
import jax.numpy as jnp
from jax import lax
from jax.experimental import pallas as pl
from jax.experimental.pallas import tpu as pltpu
from jax.experimental.pallas import tpu_sc as plsc

GAMMA = 0.1
EPS = 1e-8
N = 100000
D = 256
TOPK = 5

BLOCK = 10240
NBLK = 10
NPAD = NBLK * BLOCK
ROWS = NPAD // 128

NW = 32
BPW = NPAD // NW
LANES = 16
CHUNKS = BPW // LANES

TROWS = 8
TCOLS = N // TROWS


def _sim_body_floor(nf_ref, inj_ref, out_ref):
    out_ref[...] = jnp.zeros((1, 1, BLOCK), jnp.float32) + nf_ref[0, 0]


def _sim_body(nf_ref, inj_ref, out_ref):
    x = nf_ref[...]
    inj = inj_ref[...]
    dims = (((1,), (1,)), ((), ()))
    num = lax.dot_general(inj, x, dims,
                          preferred_element_type=jnp.float32)
    sq = lax.dot_general(jnp.ones((1, D), jnp.float32), x * x, dims,
                         preferred_element_type=jnp.float32)
    inj_norm = jnp.maximum(jnp.sqrt(jnp.sum(inj * inj)), EPS)
    sim = (-GAMMA) * (num / (jnp.maximum(jnp.sqrt(sq), EPS) * inj_norm))
    out_ref[...] = sim.reshape(1, 1, BLOCK)


def _node_sim(node_features, inj2d, interpret=False):
    return pl.pallas_call(
        _sim_body,
        grid=(NBLK,),
        in_specs=[
            pl.BlockSpec((BLOCK, D), lambda i: (i, 0)),
            pl.BlockSpec((1, D), lambda i: (0, 0)),
        ],
        out_specs=pl.BlockSpec((1, 1, BLOCK), lambda i: (i, 0, 0)),
        out_shape=jax.ShapeDtypeStruct((NBLK, 1, BLOCK), jnp.float32),
        interpret=interpret,
    )(node_features, inj2d)


def _edge_body(sim_hbm, idx_hbm, infl_hbm, out_hbm, idx_v, val_v, out_v, sem):
    wid = lax.axis_index("s") * 2 + lax.axis_index("c")
    base = pl.multiple_of(jnp.minimum(wid * BPW, N - BPW), 8)
    pltpu.sync_copy(idx_hbm.at[pl.ds(base, BPW)], idx_v)
    gather = pltpu.async_copy(sim_hbm.at[idx_v], val_v, sem)
    pltpu.sync_copy(infl_hbm.at[pl.ds(base, BPW)], out_v)
    gather.wait()

    def body(i, c):
        s = pl.ds(i * LANES, LANES)
        out_v[s] = out_v[s] + val_v[s]
        return c

    lax.fori_loop(0, CHUNKS, body, 0, unroll=4)
    pltpu.sync_copy(out_v, out_hbm.at[pl.ds(base, BPW)])


def _edge_scores(sim_flat, idx, infl):
    mesh = plsc.VectorSubcoreMesh(core_axis_name="c", subcore_axis_name="s")
    k = pl.kernel(
        _edge_body,
        out_type=jax.ShapeDtypeStruct((N,), jnp.float32),
        mesh=mesh,
        compiler_params=pltpu.CompilerParams(
            needs_layout_passes=False,
            skip_device_barrier=True,
            disable_bounds_checks=True,
            disable_semaphore_checks=True,
        ),
        scratch_types=[
            pltpu.VMEM((BPW,), jnp.int32),
            pltpu.VMEM((BPW,), jnp.float32),
            pltpu.VMEM((BPW,), jnp.float32),
            pltpu.SemaphoreType.DMA,
        ],
    )
    return k(sim_flat, idx, infl)


def _topk_body(score_ref, cand_ref, out_ref):
    s = score_ref[...]
    cand = cand_ref[...]
    r = lax.broadcasted_iota(jnp.int32, (TROWS, TCOLS), 0)
    c = lax.broadcasted_iota(jnp.int32, (TROWS, TCOLS), 1)
    flat = r * TCOLS + c
    big = jnp.int32(2**31 - 1)
    neg = jnp.float32(-jnp.inf)
    for i in range(TOPK):
        m = jnp.max(s)
        fi = jnp.min(jnp.where(s == m, flat, big))
        hit = flat == fi
        out_ref[i] = jnp.max(jnp.where(hit, cand, -1))
        s = jnp.where(hit, neg, s)


def _topk(scores2d, cand2d, interpret=False):
    return pl.pallas_call(
        _topk_body,
        in_specs=[
            pl.BlockSpec(memory_space=pltpu.VMEM),
            pl.BlockSpec(memory_space=pltpu.VMEM),
        ],
        out_specs=pl.BlockSpec(memory_space=pltpu.SMEM),
        out_shape=jax.ShapeDtypeStruct((TOPK,), jnp.int32),
        interpret=interpret,
    )(scores2d, cand2d)


def kernel(influence_scores, node_features, candidate_indices, injected_feat):
    idx = candidate_indices.astype(jnp.int32)
    sim = _node_sim(node_features, injected_feat.reshape(1, D)).reshape(NPAD)
    edge = _edge_scores(sim, idx, influence_scores)
    sel = _topk(edge.reshape(TROWS, TCOLS), idx.reshape(TROWS, TCOLS))
    return sel, edge


# speedup vs baseline: 1.0735x; 1.0222x over previous
import jax
import jax.numpy as jnp
from jax import lax
from jax.experimental import pallas as pl
from jax.experimental.pallas import tpu as pltpu
from jax.experimental.pallas import tpu_sc as plsc

GAMMA = 0.1
EPS = 1e-8
N = 100000
D = 256
TOPK = 5

BLOCK = 12544
NBLK = 8
NPAD = NBLK * BLOCK
ROWS = NPAD // 128

NW = 32
BPW = NPAD // NW
LANES = 16
CHUNKS = BPW // LANES

TROWS = 8
TCOLS = N // TROWS


def _sim_body_floor(nf_ref, inj_ref, out_ref):
    out_ref[...] = jnp.zeros((1, 1, BLOCK), jnp.float32) + nf_ref[0, 0]


def _sim_body(nf_ref, inj_ref, out_ref):
    x = nf_ref[...]
    inj = inj_ref[...]
    dims = (((1,), (1,)), ((), ()))
    num = lax.dot_general(inj, x, dims,
                          preferred_element_type=jnp.float32)
    sq = lax.dot_general(jnp.ones((1, D), jnp.float32), x * x, dims,
                         preferred_element_type=jnp.float32)
    inj_norm = jnp.maximum(jnp.sqrt(jnp.sum(inj * inj)), EPS)
    sim = (-GAMMA) * (num / (jnp.maximum(jnp.sqrt(sq), EPS) * inj_norm))
    out_ref[...] = sim.reshape(1, 1, BLOCK)


def _node_sim(node_features, inj2d, interpret=False):
    return pl.pallas_call(
        _sim_body,
        grid=(NBLK,),
        in_specs=[
            pl.BlockSpec((BLOCK, D), lambda i: (i, 0)),
            pl.BlockSpec((1, D), lambda i: (0, 0)),
        ],
        out_specs=pl.BlockSpec((1, 1, BLOCK), lambda i: (i, 0, 0)),
        out_shape=jax.ShapeDtypeStruct((NBLK, 1, BLOCK), jnp.float32),
        interpret=interpret,
    )(node_features, inj2d)


def _edge_body(sim_hbm, idx_hbm, infl_hbm, out_hbm, idx_v, val_v, out_v, sem):
    wid = lax.axis_index("s") * 2 + lax.axis_index("c")
    base = pl.multiple_of(jnp.minimum(wid * BPW, N - BPW), 8)
    pltpu.sync_copy(idx_hbm.at[pl.ds(base, BPW)], idx_v)
    gather = pltpu.async_copy(sim_hbm.at[idx_v], val_v, sem)
    pltpu.sync_copy(infl_hbm.at[pl.ds(base, BPW)], out_v)
    gather.wait()

    def body(i, c):
        s = pl.ds(i * LANES, LANES)
        out_v[s] = out_v[s] + val_v[s]
        return c

    lax.fori_loop(0, CHUNKS, body, 0, unroll=4)
    pltpu.sync_copy(out_v, out_hbm.at[pl.ds(base, BPW)])


def _edge_scores(sim_flat, idx, infl):
    mesh = plsc.VectorSubcoreMesh(core_axis_name="c", subcore_axis_name="s")
    k = pl.kernel(
        _edge_body,
        out_type=jax.ShapeDtypeStruct((N,), jnp.float32),
        mesh=mesh,
        compiler_params=pltpu.CompilerParams(
            needs_layout_passes=False,
            skip_device_barrier=True,
            disable_bounds_checks=True,
            disable_semaphore_checks=True,
        ),
        scratch_types=[
            pltpu.VMEM((BPW,), jnp.int32),
            pltpu.VMEM((BPW,), jnp.float32),
            pltpu.VMEM((BPW,), jnp.float32),
            pltpu.SemaphoreType.DMA,
        ],
    )
    return k(sim_flat, idx, infl)


def _topk_body(score_ref, cand_ref, out_ref):
    s = score_ref[...]
    cand = cand_ref[...]
    r = lax.broadcasted_iota(jnp.int32, (TROWS, TCOLS), 0)
    c = lax.broadcasted_iota(jnp.int32, (TROWS, TCOLS), 1)
    flat = r * TCOLS + c
    big = jnp.int32(2**31 - 1)
    neg = jnp.float32(-jnp.inf)
    for i in range(TOPK):
        m = jnp.max(s)
        fi = jnp.min(jnp.where(s == m, flat, big))
        hit = flat == fi
        out_ref[i] = jnp.max(jnp.where(hit, cand, -1))
        s = jnp.where(hit, neg, s)


def _topk(scores2d, cand2d, interpret=False):
    return pl.pallas_call(
        _topk_body,
        in_specs=[
            pl.BlockSpec(memory_space=pltpu.VMEM),
            pl.BlockSpec(memory_space=pltpu.VMEM),
        ],
        out_specs=pl.BlockSpec(memory_space=pltpu.SMEM),
        out_shape=jax.ShapeDtypeStruct((TOPK,), jnp.int32),
        interpret=interpret,
    )(scores2d, cand2d)


def kernel(influence_scores, node_features, candidate_indices, injected_feat):
    idx = candidate_indices.astype(jnp.int32)
    sim = _node_sim(node_features, injected_feat.reshape(1, D)).reshape(NPAD)
    edge = _edge_scores(sim, idx, influence_scores)
    sel = _topk(edge.reshape(TROWS, TCOLS), idx.reshape(TROWS, TCOLS))
    return sel, edge


# speedup vs baseline: 1.0747x; 1.0011x over previous
import jax
import jax.numpy as jnp
from jax import lax
from jax.experimental import pallas as pl
from jax.experimental.pallas import tpu as pltpu
from jax.experimental.pallas import tpu_sc as plsc

GAMMA = 0.1
EPS = 1e-8
N = 100000
D = 256
TOPK = 5

BLOCK = 12544
NBLK = 8
NPAD = NBLK * BLOCK
ROWS = NPAD // 128

NW = 32
BPW = NPAD // NW
LANES = 16
CHUNKS = BPW // LANES

TROWS = 8
TCOLS = N // TROWS


def _sim_body(nf_ref, inj_ref, out_ref):
    x = nf_ref[...]
    inj = inj_ref[...]
    dims = (((1,), (1,)), ((), ()))
    num = lax.dot_general(inj, x, dims,
                          preferred_element_type=jnp.float32)
    sq = lax.dot_general(jnp.ones((1, D), jnp.float32), x * x, dims,
                         preferred_element_type=jnp.float32)
    inj_norm = jnp.maximum(jnp.sqrt(jnp.sum(inj * inj)), EPS)
    sim = (-GAMMA) * (num / (jnp.maximum(jnp.sqrt(sq), EPS) * inj_norm))
    out_ref[...] = sim.reshape(1, 1, BLOCK)


def _node_sim(node_features, inj2d, interpret=False):
    return pl.pallas_call(
        _sim_body,
        grid=(NBLK,),
        in_specs=[
            pl.BlockSpec((BLOCK, D), lambda i: (i, 0)),
            pl.BlockSpec((1, D), lambda i: (0, 0)),
        ],
        out_specs=pl.BlockSpec((1, 1, BLOCK), lambda i: (i, 0, 0)),
        out_shape=jax.ShapeDtypeStruct((NBLK, 1, BLOCK), jnp.float32),
        interpret=interpret,
    )(node_features, inj2d)


def _edge_body(sim_hbm, idx_hbm, infl_hbm, out_hbm, idx_v, val_v, out_v, sem):
    wid = lax.axis_index("s") * 2 + lax.axis_index("c")
    base = pl.multiple_of(jnp.minimum(wid * BPW, N - BPW), 8)
    pltpu.sync_copy(idx_hbm.at[pl.ds(base, BPW)], idx_v)
    gather = pltpu.async_copy(sim_hbm.at[idx_v], val_v, sem)
    pltpu.sync_copy(infl_hbm.at[pl.ds(base, BPW)], out_v)
    gather.wait()

    def body(i, c):
        s = pl.ds(i * LANES, LANES)
        out_v[s] = out_v[s] + val_v[s]
        return c

    lax.fori_loop(0, CHUNKS, body, 0, unroll=4)
    pltpu.sync_copy(out_v, out_hbm.at[pl.ds(base, BPW)])


def _edge_scores(sim_flat, idx, infl):
    mesh = plsc.VectorSubcoreMesh(core_axis_name="c", subcore_axis_name="s")
    k = pl.kernel(
        _edge_body,
        out_type=jax.ShapeDtypeStruct((N,), jnp.float32),
        mesh=mesh,
        compiler_params=pltpu.CompilerParams(
            needs_layout_passes=False,
            skip_device_barrier=True,
            disable_bounds_checks=True,
            disable_semaphore_checks=True,
        ),
        scratch_types=[
            pltpu.VMEM((BPW,), jnp.int32),
            pltpu.VMEM((BPW,), jnp.float32),
            pltpu.VMEM((BPW,), jnp.float32),
            pltpu.SemaphoreType.DMA,
        ],
    )
    return k(sim_flat, idx, infl)


def _topk_body(score_ref, cand_ref, out_ref):
    s = score_ref[...]
    cand = cand_ref[...]
    r = lax.broadcasted_iota(jnp.int32, (TROWS, TCOLS), 0)
    c = lax.broadcasted_iota(jnp.int32, (TROWS, TCOLS), 1)
    flat = r * TCOLS + c
    big = jnp.int32(2**31 - 1)
    neg = jnp.float32(-jnp.inf)
    for i in range(TOPK):
        m = jnp.max(s)
        fi = jnp.min(jnp.where(s == m, flat, big))
        hit = flat == fi
        out_ref[i] = jnp.max(jnp.where(hit, cand, -1))
        s = jnp.where(hit, neg, s)


def _topk(scores2d, cand2d, interpret=False):
    return pl.pallas_call(
        _topk_body,
        in_specs=[
            pl.BlockSpec(memory_space=pltpu.VMEM),
            pl.BlockSpec(memory_space=pltpu.VMEM),
        ],
        out_specs=pl.BlockSpec(memory_space=pltpu.SMEM),
        out_shape=jax.ShapeDtypeStruct((TOPK,), jnp.int32),
        interpret=interpret,
    )(scores2d, cand2d)


def kernel(influence_scores, node_features, candidate_indices, injected_feat):
    idx = candidate_indices.astype(jnp.int32)
    sim = _node_sim(node_features, injected_feat.reshape(1, D)).reshape(NPAD)
    edge = _edge_scores(sim, idx, influence_scores)
    sel = _topk(edge.reshape(TROWS, TCOLS), idx.reshape(TROWS, TCOLS))
    return sel, edge
